# baseline (speedup 1.0000x reference)
import jax
import jax.numpy as jnp
from jax import lax
from jax.experimental import pallas as pl
from jax.experimental.pallas import tpu as pltpu
from jax.experimental.pallas import tpu_sc as plsc

N = 1048576
ROW = 1024
NSUB = 16
S = N // NSUB
H = S // 2
TW = 2048
NT = H // TW
VPT = TW // 16
TRASH = NSUB * TW


def _body(x_hbm, wr_hbm, wc_hbm, out_hbm, *rest):
    bufs = rest[:NT]
    x_v, wa_v, wb_v, sem = rest[NT:]
    w = lax.axis_index("s")
    lanes = lax.iota(jnp.int32, 16)

    def _z(i, _):
        wa_v[pl.ds(i * 16, 16)] = jnp.zeros((16,), jnp.float32)
        return _
    lax.fori_loop(0, TW // 16, _z, None)

    def _zcopy(i, _):
        pltpu.sync_copy(wa_v, out_hbm.at[pl.ds(w * S + i * TW, TW)])
        return _
    lax.fori_loop(0, S // TW, _zcopy, None)

    trash0 = N + w * TW

    for h in range(2):
        hbase = (h * NSUB + w) * H

        pltpu.sync_copy(x_hbm.at[pl.ds(hbase, H)], x_v)

        for t in range(NT):
            pltpu.sync_copy(wr_hbm.at[pl.ds(hbase + t * TW, TW)], wa_v)
            pltpu.sync_copy(wc_hbm.at[pl.ds(hbase + t * TW, TW)], wb_v)
            buf = bufs[t]

            def _cvt(i, _):
                v = wa_v[pl.ds(i * 16, 16)] + 1024.0 * wb_v[pl.ds(i * 16, 16)]
                buf[pl.ds(i * 16, 16)] = v.astype(jnp.int32)
                return _
            lax.fori_loop(0, VPT, _cvt, None)

        for t in range(NT):
            buf = bufs[t]

            def _red(i, _):
                e = t * VPT + i
                cur = buf[pl.ds(i * 16, 16)]
                nxt = buf[pl.ds(i * 16 + 16, 16)]
                trash = (trash0 + (e * 16) % TW) + lanes
                buf[pl.ds(i * 16, 16)] = jnp.where(cur != nxt, cur, trash)
                return _
            lax.fori_loop(0, VPT - 1, _red, None)

            if t < NT - 1:
                e = t * VPT + VPT - 1
                cur = buf[pl.ds(TW - 16, 16)]
                nxt = bufs[t + 1][pl.ds(0, 16)]
                trash = (trash0 + (e * 16) % TW) + lanes
                buf[pl.ds(TW - 16, 16)] = jnp.where(cur != nxt, cur, trash)

        plsc.subcore_barrier()
        for r in range(NSUB):
            @pl.when(w == r)
            def _fire():
                for t in range(NT):
                    pltpu.async_copy(
                        x_v.at[pl.ds(t * TW, TW)],
                        out_hbm.at[bufs[t]],
                        sem,
                    )
                for t in range(NT):
                    pltpu.make_async_copy(
                        x_hbm.at[pl.ds(0, TW)], x_v.at[pl.ds(0, TW)], sem
                    ).wait()
            plsc.subcore_barrier()


@jax.jit
def _scatter(x, wr, wc):
    mesh = plsc.VectorSubcoreMesh(
        core_axis_name="c", subcore_axis_name="s", num_cores=1
    )
    return pl.kernel(
        _body,
        out_type=jax.ShapeDtypeStruct((N + TRASH,), jnp.float32),
        mesh=mesh,
        scratch_types=(
            [pltpu.VMEM((TW,), jnp.int32) for _ in range(NT)]
            + [
                pltpu.VMEM((H,), jnp.float32),
                pltpu.VMEM((TW,), jnp.float32),
                pltpu.VMEM((TW,), jnp.float32),
                pltpu.SemaphoreType.DMA,
            ]
        ),
    )(x, wr, wc)


def kernel(x, weights_row, weights_column):
    return _scatter(x, weights_row, weights_column)[:N]


# speedup vs baseline: 27.9471x; 27.9471x over previous
import jax
import jax.numpy as jnp
from jax import lax
from jax.experimental import pallas as pl
from jax.experimental.pallas import tpu as pltpu
from jax.experimental.pallas import tpu_sc as plsc

N = 1048576
ROW = 1024
NSUB = 16
S = N // NSUB
H = S // 4
TW = 2048
NT = H // TW
VPT = TW // 16
TRASH = NSUB * TW


def _body(x_hbm, wr_hbm, wc_hbm, out_hbm, *rest):
    bufs = rest[:NT]
    shared, x_v, wa_v, wb_v, sem = rest[NT:]
    w = lax.axis_index("s")
    lanes = lax.iota(jnp.int32, 16)

    def _z(i, _):
        wa_v[pl.ds(i * 16, 16)] = jnp.zeros((16,), jnp.float32)
        return _
    lax.fori_loop(0, TW // 16, _z, None)

    def _zcopy(i, _):
        pltpu.sync_copy(wa_v, shared.at[pl.ds(w * S + i * TW, TW)])
        return _
    lax.fori_loop(0, S // TW, _zcopy, None)

    trash0 = N + w * TW

    for h in range(4):
        hbase = (h * NSUB + w) * H

        pltpu.sync_copy(x_hbm.at[pl.ds(hbase, H)], x_v)

        for t in range(NT):
            pltpu.sync_copy(wr_hbm.at[pl.ds(hbase + t * TW, TW)], wa_v)
            pltpu.sync_copy(wc_hbm.at[pl.ds(hbase + t * TW, TW)], wb_v)
            buf = bufs[t]

            def _cvt(i, _):
                v = wa_v[pl.ds(i * 16, 16)] + 1024.0 * wb_v[pl.ds(i * 16, 16)]
                buf[pl.ds(i * 16, 16)] = v.astype(jnp.int32)
                return _
            lax.fori_loop(0, VPT, _cvt, None)

        for t in range(NT):
            buf = bufs[t]

            def _red(i, _):
                e = t * VPT + i
                cur = buf[pl.ds(i * 16, 16)]
                nxt = buf[pl.ds(i * 16 + 16, 16)]
                trash = (trash0 + (e * 16) % TW) + lanes
                buf[pl.ds(i * 16, 16)] = jnp.where(cur != nxt, cur, trash)
                return _
            lax.fori_loop(0, VPT - 1, _red, None)

            if t < NT - 1:
                e = t * VPT + VPT - 1
                cur = buf[pl.ds(TW - 16, 16)]
                nxt = bufs[t + 1][pl.ds(0, 16)]
                trash = (trash0 + (e * 16) % TW) + lanes
                buf[pl.ds(TW - 16, 16)] = jnp.where(cur != nxt, cur, trash)

        plsc.subcore_barrier()
        for r in range(NSUB):
            @pl.when(w == r)
            def _fire():
                for t in range(NT):
                    pltpu.async_copy(
                        x_v.at[pl.ds(t * TW, TW)],
                        shared.at[bufs[t]],
                        sem,
                    )
                for t in range(NT):
                    pltpu.make_async_copy(
                        x_hbm.at[pl.ds(0, TW)], x_v.at[pl.ds(0, TW)], sem
                    ).wait()
            plsc.subcore_barrier()

    pltpu.sync_copy(shared.at[pl.ds(w * S, S)], out_hbm.at[pl.ds(w * S, S)])


@jax.jit
def _scatter(x, wr, wc):
    mesh = plsc.VectorSubcoreMesh(
        core_axis_name="c", subcore_axis_name="s", num_cores=1
    )
    return pl.kernel(
        _body,
        out_type=jax.ShapeDtypeStruct((N,), jnp.float32),
        mesh=mesh,
        scratch_types=(
            [pltpu.VMEM((TW,), jnp.int32) for _ in range(NT)]
            + [
                pltpu.VMEM_SHARED((N + TRASH,), jnp.float32),
                pltpu.VMEM((H,), jnp.float32),
                pltpu.VMEM((TW,), jnp.float32),
                pltpu.VMEM((TW,), jnp.float32),
                pltpu.SemaphoreType.DMA,
            ]
        ),
    )(x, wr, wc)


def kernel(x, weights_row, weights_column):
    return _scatter(x, weights_row, weights_column)


# speedup vs baseline: 30.6157x; 1.0955x over previous
import jax
import jax.numpy as jnp
from jax import lax
from jax.experimental import pallas as pl
from jax.experimental.pallas import tpu as pltpu
from jax.experimental.pallas import tpu_sc as plsc

N = 1048576
ROW = 1024
NSUB = 16
S = N // NSUB
H = S // 4
TW = 2048
NT = H // TW
VPT = TW // 16
TRASH = NSUB * TW


def _body(x_hbm, wr_hbm, wc_hbm, out_hbm, *rest):
    bufs = rest[:NT]
    shared, x_v, wa_v, wb_v, wc_v, wd_v, sem, sem_x, sem_w = rest[NT:]
    w = lax.axis_index("s")
    lanes = lax.iota(jnp.int32, 16)

    def _drain_w(n):
        for _ in range(n):
            pltpu.make_async_copy(
                x_hbm.at[pl.ds(0, TW)], wb_v, sem_w
            ).wait()

    def _z(i, _):
        wa_v[pl.ds(i * 16, 16)] = jnp.zeros((16,), jnp.float32)
        return _
    lax.fori_loop(0, TW // 16, _z, None)

    def _zcopy(i, _):
        pltpu.async_copy(wa_v, shared.at[pl.ds(w * S + i * TW, TW)], sem_w)
        return _
    lax.fori_loop(0, S // TW, _zcopy, None)

    trash0 = N + w * TW

    for h in range(4):
        hbase = (h * NSUB + w) * H

        pltpu.async_copy(x_hbm.at[pl.ds(hbase, H)], x_v, sem_x)

        if h == 0:
            _drain_w(S // TW)

        pltpu.async_copy(wr_hbm.at[pl.ds(hbase, TW)], wa_v, sem_w)
        pltpu.async_copy(wc_hbm.at[pl.ds(hbase, TW)], wb_v, sem_w)
        for t in range(NT):
            pa, pb = (wa_v, wb_v) if t % 2 == 0 else (wc_v, wd_v)
            _drain_w(2)
            if t + 1 < NT:
                na, nb = (wa_v, wb_v) if t % 2 == 1 else (wc_v, wd_v)
                pltpu.async_copy(
                    wr_hbm.at[pl.ds(hbase + (t + 1) * TW, TW)], na, sem_w)
                pltpu.async_copy(
                    wc_hbm.at[pl.ds(hbase + (t + 1) * TW, TW)], nb, sem_w)
            buf = bufs[t]

            def _cvt(i, _):
                v = pa[pl.ds(i * 16, 16)] + 1024.0 * pb[pl.ds(i * 16, 16)]
                buf[pl.ds(i * 16, 16)] = v.astype(jnp.int32)
                return _
            lax.fori_loop(0, VPT, _cvt, None)

        for t in range(NT):
            buf = bufs[t]

            def _red(i, _):
                e = t * VPT + i
                cur = buf[pl.ds(i * 16, 16)]
                nxt = buf[pl.ds(i * 16 + 16, 16)]
                trash = (trash0 + (e * 16) % TW) + lanes
                buf[pl.ds(i * 16, 16)] = jnp.where(cur != nxt, cur, trash)
                return _
            lax.fori_loop(0, VPT - 1, _red, None)

            if t < NT - 1:
                e = t * VPT + VPT - 1
                cur = buf[pl.ds(TW - 16, 16)]
                nxt = bufs[t + 1][pl.ds(0, 16)]
                trash = (trash0 + (e * 16) % TW) + lanes
                buf[pl.ds(TW - 16, 16)] = jnp.where(cur != nxt, cur, trash)

        pltpu.make_async_copy(x_hbm.at[pl.ds(0, H)], x_v, sem_x).wait()
        plsc.subcore_barrier()
        for r in range(NSUB):
            @pl.when(w == r)
            def _fire():
                for t in range(NT):
                    pltpu.async_copy(
                        x_v.at[pl.ds(t * TW, TW)],
                        shared.at[bufs[t]],
                        sem,
                    )
                for t in range(NT):
                    pltpu.make_async_copy(
                        x_hbm.at[pl.ds(0, TW)], x_v.at[pl.ds(0, TW)], sem
                    ).wait()
            plsc.subcore_barrier()

    pltpu.sync_copy(shared.at[pl.ds(w * S, S)], out_hbm.at[pl.ds(w * S, S)])


@jax.jit
def _scatter(x, wr, wc):
    mesh = plsc.VectorSubcoreMesh(
        core_axis_name="c", subcore_axis_name="s", num_cores=1
    )
    return pl.kernel(
        _body,
        out_type=jax.ShapeDtypeStruct((N,), jnp.float32),
        mesh=mesh,
        scratch_types=(
            [pltpu.VMEM((TW,), jnp.int32) for _ in range(NT)]
            + [
                pltpu.VMEM_SHARED((N + TRASH,), jnp.float32),
                pltpu.VMEM((H,), jnp.float32),
                pltpu.VMEM((TW,), jnp.float32),
                pltpu.VMEM((TW,), jnp.float32),
                pltpu.VMEM((TW,), jnp.float32),
                pltpu.VMEM((TW,), jnp.float32),
                pltpu.SemaphoreType.DMA,
                pltpu.SemaphoreType.DMA,
                pltpu.SemaphoreType.DMA,
            ]
        ),
    )(x, wr, wc)


def kernel(x, weights_row, weights_column):
    return _scatter(x, weights_row, weights_column)


# speedup vs baseline: 32.2225x; 1.0525x over previous
import jax
import jax.numpy as jnp
from jax import lax
from jax.experimental import pallas as pl
from jax.experimental.pallas import tpu as pltpu
from jax.experimental.pallas import tpu_sc as plsc

N = 1048576
ROW = 1024
NSUB = 16
S = N // NSUB
H = S // 4
TW = 2048
NT = H // TW
VPT = TW // 16
TRASH = NSUB * TW


def _body(x_hbm, wr_hbm, wc_hbm, out_hbm, *rest):
    bufs = rest[:NT]
    shared, x_v, wa_v, wb_v, wc_v, wd_v, sem, sem_x, sem_w = rest[NT:]
    w = lax.axis_index("s")
    lanes = lax.iota(jnp.int32, 16)

    def _drain_w(n):
        for _ in range(n):
            pltpu.make_async_copy(
                x_hbm.at[pl.ds(0, TW)], wb_v, sem_w
            ).wait()

    def _z(i, _):
        wa_v[pl.ds(i * 16, 16)] = jnp.zeros((16,), jnp.float32)
        return _
    lax.fori_loop(0, TW // 16, _z, None)

    def _zcopy(i, _):
        pltpu.async_copy(wa_v, shared.at[pl.ds(w * S + i * TW, TW)], sem_w)
        return _
    lax.fori_loop(0, S // TW, _zcopy, None)

    trash0 = N + w * TW

    for h in range(4):
        hbase = (h * NSUB + w) * H

        pltpu.async_copy(x_hbm.at[pl.ds(hbase, H)], x_v, sem_x)

        if h == 0:
            _drain_w(S // TW)

        pltpu.async_copy(wr_hbm.at[pl.ds(hbase, TW)], wa_v, sem_w)
        pltpu.async_copy(wc_hbm.at[pl.ds(hbase, TW)], wb_v, sem_w)
        prev = None
        for t in range(NT):
            pa, pb = (wa_v, wb_v) if t % 2 == 0 else (wc_v, wd_v)
            _drain_w(2)
            if t + 1 < NT:
                na, nb = (wa_v, wb_v) if t % 2 == 1 else (wc_v, wd_v)
                pltpu.async_copy(
                    wr_hbm.at[pl.ds(hbase + (t + 1) * TW, TW)], na, sem_w)
                pltpu.async_copy(
                    wc_hbm.at[pl.ds(hbase + (t + 1) * TW, TW)], nb, sem_w)
            buf = bufs[t]

            iv0 = (pa[pl.ds(0, 16)] + 1024.0 * pb[pl.ds(0, 16)]).astype(
                jnp.int32)
            if t > 0:
                e = t * VPT - 1
                trash = (trash0 + (e * 16) % TW) + lanes
                bufs[t - 1][pl.ds(TW - 16, 16)] = jnp.where(
                    prev != iv0, prev, trash)

            def _cvt(i, prev):
                v = pa[pl.ds(i * 16, 16)] + 1024.0 * pb[pl.ds(i * 16, 16)]
                iv = v.astype(jnp.int32)
                e = t * VPT + i - 1
                trash = (trash0 + (e * 16) % TW) + lanes
                buf[pl.ds((i - 1) * 16, 16)] = jnp.where(
                    prev != iv, prev, trash)
                return iv
            prev = lax.fori_loop(1, VPT, _cvt, iv0)

        bufs[NT - 1][pl.ds(TW - 16, 16)] = prev

        pltpu.make_async_copy(x_hbm.at[pl.ds(0, H)], x_v, sem_x).wait()
        plsc.subcore_barrier()
        for r in range(NSUB):
            @pl.when(w == r)
            def _fire():
                for t in range(NT):
                    pltpu.async_copy(
                        x_v.at[pl.ds(t * TW, TW)],
                        shared.at[bufs[t]],
                        sem,
                    )
                for t in range(NT):
                    pltpu.make_async_copy(
                        x_hbm.at[pl.ds(0, TW)], x_v.at[pl.ds(0, TW)], sem
                    ).wait()
            plsc.subcore_barrier()

    pltpu.sync_copy(shared.at[pl.ds(w * S, S)], out_hbm.at[pl.ds(w * S, S)])


@jax.jit
def _scatter(x, wr, wc):
    mesh = plsc.VectorSubcoreMesh(
        core_axis_name="c", subcore_axis_name="s", num_cores=1
    )
    return pl.kernel(
        _body,
        out_type=jax.ShapeDtypeStruct((N,), jnp.float32),
        mesh=mesh,
        scratch_types=(
            [pltpu.VMEM((TW,), jnp.int32) for _ in range(NT)]
            + [
                pltpu.VMEM_SHARED((N + TRASH,), jnp.float32),
                pltpu.VMEM((H,), jnp.float32),
                pltpu.VMEM((TW,), jnp.float32),
                pltpu.VMEM((TW,), jnp.float32),
                pltpu.VMEM((TW,), jnp.float32),
                pltpu.VMEM((TW,), jnp.float32),
                pltpu.SemaphoreType.DMA,
                pltpu.SemaphoreType.DMA,
                pltpu.SemaphoreType.DMA,
            ]
        ),
    )(x, wr, wc)


def kernel(x, weights_row, weights_column):
    return _scatter(x, weights_row, weights_column)


# speedup vs baseline: 33.1272x; 1.0281x over previous
import jax
import jax.numpy as jnp
from jax import lax
from jax.experimental import pallas as pl
from jax.experimental.pallas import tpu as pltpu
from jax.experimental.pallas import tpu_sc as plsc

N = 1048576
ROW = 1024
NSUB = 16
S = N // NSUB
H = S // 4
TW = 4096
NT = H // TW
VPT = TW // 16
TRASH = NSUB * TW


def _body(x_hbm, wr_hbm, wc_hbm, out_hbm, *rest):
    bufs = rest[:NT]
    shared, x_v, wa_v, wb_v, wc_v, wd_v, sem, sem_x, sem_w = rest[NT:]
    w = lax.axis_index("s")
    lanes = lax.iota(jnp.int32, 16)

    def _drain_w(n):
        for _ in range(n):
            pltpu.make_async_copy(
                x_hbm.at[pl.ds(0, TW)], wb_v, sem_w
            ).wait()

    def _z(i, _):
        wa_v[pl.ds(i * 16, 16)] = jnp.zeros((16,), jnp.float32)
        return _
    lax.fori_loop(0, TW // 16, _z, None)

    def _zcopy(i, _):
        pltpu.async_copy(wa_v, shared.at[pl.ds(w * S + i * TW, TW)], sem_w)
        return _
    lax.fori_loop(0, S // TW, _zcopy, None)

    trash0 = N + w * TW

    for h in range(4):
        hbase = (h * NSUB + w) * H

        pltpu.async_copy(x_hbm.at[pl.ds(hbase, H)], x_v, sem_x)

        if h == 0:
            _drain_w(S // TW)

        pltpu.async_copy(wr_hbm.at[pl.ds(hbase, TW)], wa_v, sem_w)
        pltpu.async_copy(wc_hbm.at[pl.ds(hbase, TW)], wb_v, sem_w)
        prev = None
        for t in range(NT):
            pa, pb = (wa_v, wb_v) if t % 2 == 0 else (wc_v, wd_v)
            _drain_w(2)
            if t + 1 < NT:
                na, nb = (wa_v, wb_v) if t % 2 == 1 else (wc_v, wd_v)
                pltpu.async_copy(
                    wr_hbm.at[pl.ds(hbase + (t + 1) * TW, TW)], na, sem_w)
                pltpu.async_copy(
                    wc_hbm.at[pl.ds(hbase + (t + 1) * TW, TW)], nb, sem_w)
            buf = bufs[t]

            iv0 = (pa[pl.ds(0, 16)] + 1024.0 * pb[pl.ds(0, 16)]).astype(
                jnp.int32)
            if t > 0:
                e = t * VPT - 1
                trash = (trash0 + (e * 16) % TW) + lanes
                bufs[t - 1][pl.ds(TW - 16, 16)] = jnp.where(
                    prev != iv0, prev, trash)

            def _cvt(i, prev):
                v = pa[pl.ds(i * 16, 16)] + 1024.0 * pb[pl.ds(i * 16, 16)]
                iv = v.astype(jnp.int32)
                e = t * VPT + i - 1
                trash = (trash0 + (e * 16) % TW) + lanes
                buf[pl.ds((i - 1) * 16, 16)] = jnp.where(
                    prev != iv, prev, trash)
                return iv
            prev = lax.fori_loop(1, VPT, _cvt, iv0)

        bufs[NT - 1][pl.ds(TW - 16, 16)] = prev

        pltpu.make_async_copy(x_hbm.at[pl.ds(0, H)], x_v, sem_x).wait()
        plsc.subcore_barrier()
        for r in range(NSUB):
            @pl.when(w == r)
            def _fire():
                for t in range(NT):
                    pltpu.async_copy(
                        x_v.at[pl.ds(t * TW, TW)],
                        shared.at[bufs[t]],
                        sem,
                    )
                for t in range(NT):
                    pltpu.make_async_copy(
                        x_hbm.at[pl.ds(0, TW)], x_v.at[pl.ds(0, TW)], sem
                    ).wait()
            plsc.subcore_barrier()

    pltpu.sync_copy(shared.at[pl.ds(w * S, S)], out_hbm.at[pl.ds(w * S, S)])


@jax.jit
def _scatter(x, wr, wc):
    mesh = plsc.VectorSubcoreMesh(
        core_axis_name="c", subcore_axis_name="s", num_cores=1
    )
    return pl.kernel(
        _body,
        out_type=jax.ShapeDtypeStruct((N,), jnp.float32),
        mesh=mesh,
        scratch_types=(
            [pltpu.VMEM((TW,), jnp.int32) for _ in range(NT)]
            + [
                pltpu.VMEM_SHARED((N + TRASH,), jnp.float32),
                pltpu.VMEM((H,), jnp.float32),
                pltpu.VMEM((TW,), jnp.float32),
                pltpu.VMEM((TW,), jnp.float32),
                pltpu.VMEM((TW,), jnp.float32),
                pltpu.VMEM((TW,), jnp.float32),
                pltpu.SemaphoreType.DMA,
                pltpu.SemaphoreType.DMA,
                pltpu.SemaphoreType.DMA,
            ]
        ),
    )(x, wr, wc)


def kernel(x, weights_row, weights_column):
    return _scatter(x, weights_row, weights_column)
